# baseline (speedup 1.0000x reference)
import dataclasses

import jax
import jax.numpy as jnp
from jax import lax
from jax.experimental import pallas as pl
from jax.experimental.pallas import tpu as pltpu
from jax.experimental.pallas import tpu_sc as plsc

NN = 100000
NPAD = 100096
HD = 16
NE = 1600000
NT = 16
B = 512
IDXW = 128
NSUB = B // IDXW
EPT = 100352
EPAD = NT * EPT
NCHUNK = EPT // B
NCHT = EPAD // B
ZR = NPAD // NT
ZFULL = ZR // B
ZREM = ZR - ZFULL * B

_mesh = plsc.VectorSubcoreMesh(core_axis_name="c", subcore_axis_name="s")


def _layer_body(emb, sdv, out, sdvx, rows, acc, semi, semz,
                semg0, semg1, sems0, sems1):
    c = lax.axis_index("c")
    s = lax.axis_index("s")
    semg = (semg0, semg1)
    sems = (sems0, sems1)

    @pl.loop(0, B)
    def _(i):
        rows[0, i] = jnp.zeros((HD,), jnp.float32)

    zbase = s * ZR

    def zero_cps():
        cps = [pltpu.make_async_copy(rows.at[0],
                                     acc.at[pl.ds(zbase + z * B, B)], semz)
               for z in range(ZFULL)]
        cps.append(pltpu.make_async_copy(rows.at[0, pl.ds(0, ZREM)],
                                         acc.at[pl.ds(zbase + ZFULL * B,
                                                      ZREM)], semz))
        return cps

    for cp in zero_cps():
        cp.start()
    for cp in zero_cps():
        cp.wait()
    plsc.subcore_barrier()

    def idx_cp(g, b):
        return pltpu.make_async_copy(sdv.at[c, s * NCHUNK + g],
                                     sdvx.at[b], semi)

    def gather_cps(b):
        return [
            pltpu.make_async_copy(emb.at[sdvx.at[b, 0, j]],
                                  rows.at[b, pl.ds(j * IDXW, IDXW)], semg[b])
            for j in range(NSUB)
        ]

    def scatter_cps(b):
        return [
            pltpu.make_async_copy(rows.at[b, pl.ds(j * IDXW, IDXW)],
                                  acc.at[sdvx.at[b, 1, j]], sems[b])
            for j in range(NSUB)
        ]

    def prep(g, b):
        idx_cp(g, b).start()
        idx_cp(g, b).wait()
        for cp in gather_cps(b):
            cp.start()

    prep(0, 0)

    @pl.loop(0, NCHUNK // 2)
    def _(t):
        for b in range(2):
            g = 2 * t + b
            for cp in gather_cps(b):
                cp.wait()

            def bracket(wait_prev):
                def go():
                    if wait_prev:
                        for cp in scatter_cps(1 - b):
                            cp.wait()
                    prep(g + 1, 1 - b)
                return go

            if b == 0:
                pl.when(t > 0)(bracket(True))
                pl.when(t == 0)(bracket(False))
            else:
                pl.when(t < NCHUNK // 2 - 1)(bracket(True))

            for j in range(NSUB):
                @pl.loop(0, IDXW // 16)
                def _(q):
                    v16 = plsc.bitcast(sdvx[b, 2, j, pl.ds(q * 16, 16)],
                                       jnp.float32)
                    for i in range(16):
                        e = j * IDXW + q * 16 + i
                        rows[b, e] = rows[b, e] * v16[i]

            for cp in scatter_cps(b):
                cp.start(add=True)

    for cp in scatter_cps(0):
        cp.wait()
    for cp in scatter_cps(1):
        cp.wait()
    plsc.subcore_barrier()

    ob = c * NPAD + s * ZR

    def wb_cps():
        cps = [pltpu.make_async_copy(acc.at[pl.ds(zbase + z * B, B)],
                                     out.at[pl.ds(ob + z * B, B)], semz)
               for z in range(ZFULL)]
        cps.append(pltpu.make_async_copy(
            acc.at[pl.ds(zbase + ZFULL * B, ZREM)],
            out.at[pl.ds(ob + ZFULL * B, ZREM)], semz))
        return cps

    for cp in wb_cps():
        cp.start()
    for cp in wb_cps():
        cp.wait()


_layer = pl.kernel(
    _layer_body,
    out_type=jax.ShapeDtypeStruct((2 * NPAD, HD), jnp.float32),
    mesh=_mesh,
    compiler_params=(
        pltpu.CompilerParams(use_tc_tiling_on_sc=False,
                             needs_layout_passes=False)
        if "needs_layout_passes" in pltpu.CompilerParams.__dataclass_fields__
        else pltpu.CompilerParams(use_tc_tiling_on_sc=False)),
    scratch_types=[
        pltpu.VMEM((2, 3, NSUB, IDXW), jnp.int32),
        pltpu.VMEM((2, B, HD), jnp.float32),
        pltpu.VMEM_SHARED((NPAD, HD), jnp.float32),
        pltpu.SemaphoreType.DMA,
        pltpu.SemaphoreType.DMA,
        pltpu.SemaphoreType.DMA,
        pltpu.SemaphoreType.DMA,
        pltpu.SemaphoreType.DMA,
        pltpu.SemaphoreType.DMA,
    ],
)


def _mean_body(a0, b0, a1, b1, a2, b2, a3, b3, o):
    sl = (a0[0] + a1[0] + a2[0] + a3[0]) * 0.25
    sr = (b0[0] + b1[0] + b2[0] + b3[0]) * 0.25
    o[...] = jnp.concatenate([sl, sr], axis=1)


def _mean4(e0, e1, e2, e3):
    bn = 4000
    r = lambda x: x.reshape(2, NPAD, HD)
    in_l = pl.BlockSpec((1, bn, HD), lambda i: (0, i, 0))
    in_r = pl.BlockSpec((1, bn, HD), lambda i: (1, i, 0))
    call = pl.pallas_call(
        _mean_body,
        grid=(NN // bn,),
        in_specs=[in_l, in_r] * 4,
        out_specs=pl.BlockSpec((bn, 2 * HD), lambda i: (i, 0)),
        out_shape=jax.ShapeDtypeStruct((NN, 2 * HD), jnp.float32),
    )
    return call(r(e0), r(e0), r(e1), r(e1), r(e2), r(e2), r(e3), r(e3))


def kernel(user_emb, item_emb, adj_indices, adj_values):
    n_users = user_emb.shape[0]
    dst = adj_indices[0].astype(jnp.int32)
    src = adj_indices[1].astype(jnp.int32)
    val = adj_values.astype(jnp.float32)

    pad = EPAD - NE
    pad_idx = (jnp.arange(pad, dtype=jnp.int32) * 17) % NN
    src_p = jnp.concatenate([src, pad_idx])
    dst_p = jnp.concatenate([dst, pad_idx])
    val_p = jnp.concatenate([val, jnp.zeros((pad,), jnp.float32)])

    valr = lax.bitcast_convert_type(val_p, jnp.int32)
    blk = lambda x: x.reshape(NCHT, 1, NSUB, IDXW)
    pack = lambda sc_src: jnp.concatenate(
        [blk(sc_src), blk(dst_p), blk(valr)], axis=1)
    sdv = jnp.stack([pack(src_p), pack(src_p + NPAD)], axis=0)

    all_emb = jnp.concatenate([user_emb, item_emb], axis=0)
    row_pad = ((0, NPAD - NN), (0, 0))
    e0 = jnp.concatenate([jnp.pad(all_emb[:, :HD], row_pad),
                          jnp.pad(all_emb[:, HD:], row_pad)], axis=0)

    e1 = _layer(e0, sdv)
    e2 = _layer(e1, sdv)
    e3 = _layer(e2, sdv)

    final = _mean4(e0, e1, e2, e3)
    return final[:n_users], final[n_users:]
